# baseline (speedup 1.0000x reference)
import functools

import jax
import jax.numpy as jnp
from jax.experimental import pallas as pl
from jax.experimental.pallas import tpu as pltpu


def _moe_body(idx_ref, wv_ref, x_hbm, wr_ref, ws_ref, bs_ref, br_ref,
              out_hbm, x_buf, o_buf, in_sems, out_sems, *, nb, nsteps):
    i = pl.program_id(0)

    def in_copy(step):
        return pltpu.make_async_copy(
            x_hbm.at[pl.ds(step * nb, nb)], x_buf.at[step], in_sems.at[step]
        )

    def out_copy(step):
        return pltpu.make_async_copy(
            o_buf.at[step], out_hbm.at[pl.ds(step * nb, nb)],
            out_sems.at[step]
        )

    @pl.when(i == 0)
    def _():
        for k in range(nsteps):
            in_copy(k).start()

    in_copy(i).wait()

    for j in range(nb):
        e = idx_ref[i * nb + j]
        w = wv_ref[i * nb + j]
        wc = ws_ref[...] + w.astype(jnp.bfloat16) * wr_ref[e]
        acc = jnp.dot(wc, x_buf[i, j].astype(jnp.bfloat16),
                      preferred_element_type=jnp.float32)
        o_buf[i, j] = acc + (bs_ref[...] + w * br_ref[e])

    out_copy(i).start()

    @pl.when(i == nsteps - 1)
    def _():
        for k in range(nsteps):
            out_copy(k).wait()


def kernel(x, weights, indices, Ws, bs, Wr, br):
    B, C, H, W = x.shape
    E, O, _ = Wr.shape
    HW = H * W
    nb = 8
    nsteps = B // nb

    xf = x.reshape(B, C, HW)
    idx = indices.reshape(-1).astype(jnp.int32)
    wv = weights.reshape(-1).astype(jnp.float32)
    wr16 = Wr.astype(jnp.bfloat16)
    ws16 = Ws.astype(jnp.bfloat16)
    bs2 = bs.reshape(O, 1)
    br2 = br.reshape(E, O, 1)

    grid_spec = pltpu.PrefetchScalarGridSpec(
        num_scalar_prefetch=2,
        grid=(nsteps,),
        in_specs=[
            pl.BlockSpec(memory_space=pl.ANY),
            pl.BlockSpec((E, O, C), lambda b, i, w: (0, 0, 0)),
            pl.BlockSpec((O, C), lambda b, i, w: (0, 0)),
            pl.BlockSpec((O, 1), lambda b, i, w: (0, 0)),
            pl.BlockSpec((E, O, 1), lambda b, i, w: (0, 0, 0)),
        ],
        out_specs=pl.BlockSpec(memory_space=pl.ANY),
        scratch_shapes=[
            pltpu.VMEM((B // nb, nb, C, HW), jnp.float32),
            pltpu.VMEM((B // nb, nb, O, HW), jnp.float32),
            pltpu.SemaphoreType.DMA((B // nb,)),
            pltpu.SemaphoreType.DMA((B // nb,)),
        ],
    )
    out = pl.pallas_call(
        functools.partial(_moe_body, nb=nb, nsteps=nsteps),
        grid_spec=grid_spec,
        out_shape=jax.ShapeDtypeStruct((B, O, HW), jnp.float32),
    )(idx, wv, xf, wr16, ws16, bs2, br2)
    return out.reshape(B, O, H, W)


# speedup vs baseline: 1.0791x; 1.0791x over previous
import functools

import jax
import jax.numpy as jnp
from jax.experimental import pallas as pl
from jax.experimental.pallas import tpu as pltpu


def _moe_body(idx_ref, wv_ref, x_hbm, wr_ref, ws_ref, bs_ref, br_ref,
              out_hbm, x_buf, o_buf, in_sems, out_sems, *, nb, nsteps):
    i = pl.program_id(0)

    def in_copy(step):
        return pltpu.make_async_copy(
            x_hbm.at[pl.ds(step * nb, nb)], x_buf.at[step], in_sems.at[step]
        )

    def out_copy(step):
        return pltpu.make_async_copy(
            o_buf.at[step], out_hbm.at[pl.ds(step * nb, nb)],
            out_sems.at[step]
        )

    @pl.when(i == 0)
    def _():
        for k in range(nsteps):
            in_copy(k).start()

    in_copy(i).wait()

    for j in range(nb):
        e = idx_ref[i * nb + j]
        w = wv_ref[i * nb + j]
        wc = (ws_ref[...] + w * wr_ref[e]).astype(jnp.bfloat16)
        acc = jnp.dot(wc, x_buf[i, j].astype(jnp.bfloat16),
                      preferred_element_type=jnp.float32)
        o_buf[i, j] = acc + (bs_ref[...] + w * br_ref[e])

    out_copy(i).start()

    @pl.when(i == nsteps - 1)
    def _():
        for k in range(nsteps):
            out_copy(k).wait()


def kernel(x, weights, indices, Ws, bs, Wr, br):
    B, C, H, W = x.shape
    E, O, _ = Wr.shape
    HW = H * W
    nb = 8
    nsteps = B // nb

    xf = x.reshape(B, C, HW)
    idx = indices.reshape(-1).astype(jnp.int32)
    wv = weights.reshape(-1).astype(jnp.float32)
    bs2 = bs.reshape(O, 1)
    br2 = br.reshape(E, O, 1)

    grid_spec = pltpu.PrefetchScalarGridSpec(
        num_scalar_prefetch=2,
        grid=(nsteps,),
        in_specs=[
            pl.BlockSpec(memory_space=pl.ANY),
            pl.BlockSpec((E, O, C), lambda b, i, w: (0, 0, 0)),
            pl.BlockSpec((O, C), lambda b, i, w: (0, 0)),
            pl.BlockSpec((O, 1), lambda b, i, w: (0, 0)),
            pl.BlockSpec((E, O, 1), lambda b, i, w: (0, 0, 0)),
        ],
        out_specs=pl.BlockSpec(memory_space=pl.ANY),
        scratch_shapes=[
            pltpu.VMEM((B // nb, nb, C, HW), jnp.float32),
            pltpu.VMEM((B // nb, nb, O, HW), jnp.float32),
            pltpu.SemaphoreType.DMA((B // nb,)),
            pltpu.SemaphoreType.DMA((B // nb,)),
        ],
    )
    out = pl.pallas_call(
        functools.partial(_moe_body, nb=nb, nsteps=nsteps),
        grid_spec=grid_spec,
        out_shape=jax.ShapeDtypeStruct((B, O, HW), jnp.float32),
    )(idx, wv, xf, Wr, Ws, bs2, br2)
    return out.reshape(B, O, H, W)
